# baseline (speedup 1.0000x reference)
import jax
import jax.numpy as jnp
from jax import lax
from jax.experimental import pallas as pl
from jax.experimental.pallas import tpu as pltpu
from jax.experimental.pallas import tpu_sc as plsc

B, D, C, K = 1024, 256, 128, 10
BB = 128
NEG = -1e30

NW = 32
RPW = B // NW


def _tc_body(x_ref, retr_ref, pw_ref, pb_ref, aw_ref, gs_ref, att_ref):
    x = x_ref[...]
    logits = jnp.dot(x, pw_ref[...], preferred_element_type=jnp.float32)
    logits = logits + pb_ref[...][None, :]
    m = jnp.max(logits, axis=1, keepdims=True)
    e = jnp.exp(logits - m)
    g = e / jnp.sum(e, axis=1, keepdims=True)

    q = jnp.dot(x, aw_ref[...], preferred_element_type=jnp.float32)
    s0 = jnp.sum(q * x, axis=1)
    sk = jnp.sum(retr_ref[...] * q[None, :, :], axis=2)
    scores = jnp.concatenate([s0[None, :], sk], axis=0)

    sm = jnp.max(scores, axis=0, keepdims=True)
    se = jnp.exp(scores - sm)
    att = se * (jnp.float32(C) / jnp.sum(se, axis=0, keepdims=True))

    gs_ref[...] = att[0][:, None] * g
    att_ref[...] = jnp.concatenate(
        [att[1:], jnp.zeros((16 - K, BB), jnp.float32)], axis=0)


def _sc_body(att_hbm, y_hbm, gs_hbm, out_hbm, a_v, y_v, o_v, sem):
    wid = lax.axis_index("s") * 2 + lax.axis_index("c")
    base = wid * RPW
    ab = pl.multiple_of((wid // 4) * 128, 128)
    cb = (wid % 4) * RPW
    cp_a = pltpu.make_async_copy(att_hbm.at[:, pl.ds(ab, 128)], a_v, sem)
    cp_y = pltpu.make_async_copy(y_hbm.at[:, pl.ds(ab, 128)], y_v, sem)
    cp_g = pltpu.make_async_copy(gs_hbm.at[pl.ds(base, RPW)], o_v, sem)
    cp_a.start(); cp_y.start(); cp_g.start()
    cp_a.wait(); cp_y.wait(); cp_g.wait()

    lanes = lax.iota(jnp.int32, 16)
    for r in range(RPW // 16):
        rows = r * 16 + lanes
        for k in range(K):
            sl = pl.ds(cb + r * 16, 16)
            plsc.addupdate_scatter(o_v, [rows, y_v[k, sl]], a_v[k, sl])

    cp_o = pltpu.make_async_copy(o_v, out_hbm.at[pl.ds(base, RPW)], sem)
    cp_o.start(); cp_o.wait()


@jax.jit
def _run(graph_embeddings, retrieval_embeddings, pred_W, pred_b, adapter_W,
         retrieval_y):
    gs, att = pl.pallas_call(
        _tc_body,
        grid=(B // BB,),
        in_specs=[
            pl.BlockSpec((BB, D), lambda i: (i, 0)),
            pl.BlockSpec((K, BB, D), lambda i: (0, i, 0)),
            pl.BlockSpec((D, C), lambda i: (0, 0)),
            pl.BlockSpec((C,), lambda i: (0,)),
            pl.BlockSpec((D, D), lambda i: (0, 0)),
        ],
        out_specs=[
            pl.BlockSpec((BB, C), lambda i: (i, 0)),
            pl.BlockSpec((16, BB), lambda i: (0, i)),
        ],
        out_shape=[
            jax.ShapeDtypeStruct((B, C), jnp.float32),
            jax.ShapeDtypeStruct((16, B), jnp.float32),
        ],
        compiler_params=pltpu.CompilerParams(skip_device_barrier=True),
    )(graph_embeddings, retrieval_embeddings, pred_W, pred_b, adapter_W)

    mesh = plsc.VectorSubcoreMesh(core_axis_name="c", subcore_axis_name="s")
    fuse = pl.kernel(
        _sc_body,
        out_type=jax.ShapeDtypeStruct((B, C), jnp.float32),
        mesh=mesh,
        compiler_params=pltpu.CompilerParams(needs_layout_passes=False,
                                             skip_device_barrier=True),
        scratch_types=[
            pltpu.VMEM((16, 128), jnp.float32),
            pltpu.VMEM((K, 128), jnp.int32),
            pltpu.VMEM((RPW, C), jnp.float32),
            pltpu.SemaphoreType.DMA,
        ],
    )
    return fuse(att, retrieval_y, gs)


def kernel(graph_embeddings, retrieval_embeddings, pred_W, pred_b, adapter_W,
           retrieval_y):
    return _run(graph_embeddings, retrieval_embeddings, pred_W, pred_b,
                adapter_W, retrieval_y.astype(jnp.int32))


# speedup vs baseline: 1.1343x; 1.1343x over previous
import jax
import jax.numpy as jnp
from jax import lax
from jax.experimental import pallas as pl
from jax.experimental.pallas import tpu as pltpu
from jax.experimental.pallas import tpu_sc as plsc

B, D, C, K = 1024, 256, 128, 10
BB = 512
NEG = -1e30

NW = 32
RPW = B // NW


def _tc_body(x_ref, retr_ref, pw_ref, pb_ref, aw_ref, gs_ref, att_ref):
    x = x_ref[...]
    logits = jnp.dot(x, pw_ref[...], preferred_element_type=jnp.float32)
    logits = logits + pb_ref[...][None, :]
    m = jnp.max(logits, axis=1, keepdims=True)
    e = jnp.exp(logits - m)
    g = e / jnp.sum(e, axis=1, keepdims=True)

    q = jnp.dot(x, aw_ref[...], preferred_element_type=jnp.float32)
    s0 = jnp.sum(q * x, axis=1)
    sk = jnp.sum(retr_ref[...] * q[None, :, :], axis=2)
    scores = jnp.concatenate([s0[None, :], sk], axis=0)

    sm = jnp.max(scores, axis=0, keepdims=True)
    se = jnp.exp(scores - sm)
    att = se * (jnp.float32(C) / jnp.sum(se, axis=0, keepdims=True))

    gs_ref[...] = att[0][:, None] * g
    att_ref[...] = jnp.concatenate(
        [att[1:], jnp.zeros((16 - K, BB), jnp.float32)], axis=0)


def _sc_body(att_hbm, y_hbm, gs_hbm, out_hbm, a_v, y_v, o_v, sem):
    wid = lax.axis_index("s") * 2 + lax.axis_index("c")
    base = wid * RPW
    ab = pl.multiple_of((wid // 4) * 128, 128)
    cb = (wid % 4) * RPW
    cp_a = pltpu.make_async_copy(att_hbm.at[:, pl.ds(ab, 128)], a_v, sem)
    cp_y = pltpu.make_async_copy(y_hbm.at[:, pl.ds(ab, 128)], y_v, sem)
    cp_g = pltpu.make_async_copy(gs_hbm.at[pl.ds(base, RPW)], o_v, sem)
    cp_a.start(); cp_y.start(); cp_g.start()
    cp_a.wait(); cp_y.wait(); cp_g.wait()

    lanes = lax.iota(jnp.int32, 16)
    for r in range(RPW // 16):
        rows = r * 16 + lanes
        for k in range(K):
            sl = pl.ds(cb + r * 16, 16)
            plsc.addupdate_scatter(o_v, [rows, y_v[k, sl]], a_v[k, sl])

    cp_o = pltpu.make_async_copy(o_v, out_hbm.at[pl.ds(base, RPW)], sem)
    cp_o.start(); cp_o.wait()


@jax.jit
def _run(graph_embeddings, retrieval_embeddings, pred_W, pred_b, adapter_W,
         retrieval_y):
    gs, att = pl.pallas_call(
        _tc_body,
        grid=(B // BB,),
        in_specs=[
            pl.BlockSpec((BB, D), lambda i: (i, 0)),
            pl.BlockSpec((K, BB, D), lambda i: (0, i, 0)),
            pl.BlockSpec((D, C), lambda i: (0, 0)),
            pl.BlockSpec((C,), lambda i: (0,)),
            pl.BlockSpec((D, D), lambda i: (0, 0)),
        ],
        out_specs=[
            pl.BlockSpec((BB, C), lambda i: (i, 0)),
            pl.BlockSpec((16, BB), lambda i: (0, i)),
        ],
        out_shape=[
            jax.ShapeDtypeStruct((B, C), jnp.float32),
            jax.ShapeDtypeStruct((16, B), jnp.float32),
        ],
        compiler_params=pltpu.CompilerParams(skip_device_barrier=True),
    )(graph_embeddings, retrieval_embeddings, pred_W, pred_b, adapter_W)

    mesh = plsc.VectorSubcoreMesh(core_axis_name="c", subcore_axis_name="s")
    fuse = pl.kernel(
        _sc_body,
        out_type=jax.ShapeDtypeStruct((B, C), jnp.float32),
        mesh=mesh,
        compiler_params=pltpu.CompilerParams(needs_layout_passes=False,
                                             skip_device_barrier=True),
        scratch_types=[
            pltpu.VMEM((16, 128), jnp.float32),
            pltpu.VMEM((K, 128), jnp.int32),
            pltpu.VMEM((RPW, C), jnp.float32),
            pltpu.SemaphoreType.DMA,
        ],
    )
    return fuse(att, retrieval_y, gs)


def kernel(graph_embeddings, retrieval_embeddings, pred_W, pred_b, adapter_W,
           retrieval_y):
    return _run(graph_embeddings, retrieval_embeddings, pred_W, pred_b,
                adapter_W, retrieval_y.astype(jnp.int32))
